# baseline (speedup 1.0000x reference)
import functools

import jax
import jax.numpy as jnp
from jax import lax
from jax.experimental import pallas as pl
from jax.experimental.pallas import tpu as pltpu
from jax.experimental.pallas import tpu_sc as plsc

D = 1024
E = 8
K = 2
T = 2048
DFF = 4 * D

M = 512
HM = 256
F = 1024
FC = DFF // F
NT = (T * K) // M + E
R = NT * M

TB = 256

NC = 2
NS = 16
NW = NC * NS
LANES = 16

def _sc_mesh():
    return plsc.VectorSubcoreMesh(
        core_axis_name="c", subcore_axis_name="s",
        num_cores=NC, num_subcores=NS)


def _router_body(x_ref, g_ref, b_ref, wr_ref, xn_ref, p_ref):
    x = x_ref[...]
    mu = jnp.mean(x, axis=1, keepdims=True)
    var = jnp.mean((x - mu) ** 2, axis=1, keepdims=True)
    xn = (x - mu) * lax.rsqrt(var + 1e-5) * g_ref[...] + b_ref[...]
    xn_ref[...] = xn.astype(jnp.bfloat16)
    logits = lax.dot_general(
        xn, wr_ref[...], (((1,), (1,)), ((), ())),
        preferred_element_type=jnp.float32,
    )
    z = logits * (1.0 / 1.5)
    z = z - jnp.max(z, axis=1, keepdims=True)
    ez = jnp.exp(z)
    p_ref[...] = ez / jnp.sum(ez, axis=1, keepdims=True)


def _router(x2d, ln_g, ln_b, Wr):
    return pl.pallas_call(
        _router_body,
        grid=(T // TB,),
        in_specs=[
            pl.BlockSpec((TB, D), lambda t: (t, 0)),
            pl.BlockSpec((1, D), lambda t: (0, 0)),
            pl.BlockSpec((1, D), lambda t: (0, 0)),
            pl.BlockSpec((E, D), lambda t: (0, 0)),
        ],
        out_specs=[
            pl.BlockSpec((TB, D), lambda t: (t, 0)),
            pl.BlockSpec((TB, E), lambda t: (t, 0)),
        ],
        out_shape=[
            jax.ShapeDtypeStruct((T, D), jnp.bfloat16),
            jax.ShapeDtypeStruct((T, E), jnp.float32),
        ],
    )(x2d, ln_g.reshape(1, D), ln_b.reshape(1, D), Wr)


def _dispatch_metadata(probs):
    w, idx = lax.top_k(probs, K)
    flat_e = idx.reshape(-1)

    onehot = (flat_e[:, None] == jnp.arange(E)[None, :]).astype(jnp.int32)
    csum = jnp.cumsum(onehot, axis=0)
    counts = csum[-1]
    rank = jnp.take_along_axis(csum - onehot, flat_e[:, None], axis=1)[:, 0]

    tiles_e = (counts + M - 1) // M
    tile_start = jnp.concatenate([jnp.zeros((1,), jnp.int32),
                                  jnp.cumsum(tiles_e).astype(jnp.int32)])
    pad_start = tile_start[:E] * M

    row_of_flat = (pad_start[flat_e] + rank).astype(jnp.int32)
    r01 = row_of_flat.reshape(T, K)

    n_act = tile_start[E]
    raw_tile_e = jnp.searchsorted(
        tile_start[1:], jnp.arange(NT, dtype=jnp.int32), side="right"
    ).astype(jnp.int32)
    active = (jnp.arange(NT, dtype=jnp.int32) < n_act).astype(jnp.int32)
    last_e = raw_tile_e[jnp.maximum(n_act - 1, 0)]
    tile_e = jnp.where(active == 1, jnp.minimum(raw_tile_e, E - 1), last_e)
    blk = jnp.where(active == 1, jnp.arange(NT, dtype=jnp.int32),
                    jnp.maximum(n_act - 1, 0)).astype(jnp.int32)
    local = jnp.arange(NT, dtype=jnp.int32) - tile_start[
        jnp.minimum(raw_tile_e, E - 1)]
    used = jnp.clip(counts[jnp.minimum(raw_tile_e, E - 1)] - local * M, 0, M)
    n_half = jnp.where(active == 1, (used + HM - 1) // HM, 0).astype(jnp.int32)

    wh = w.astype(jnp.bfloat16)
    wl = (w - wh.astype(jnp.float32)).astype(jnp.bfloat16)
    w0cat = jnp.stack([wh[:, 0], wl[:, 0]], axis=1).reshape(1, T, 2)
    w1cat = jnp.stack([wh[:, 1], wl[:, 1]], axis=1).reshape(1, T, 2)
    return (w0cat, w1cat, r01[:, 0], r01[:, 1], tile_e, active, blk, n_half)


def _mm_body(te_ref, act_ref, blk_ref, nh_ref, xnb_ref, r0_ref, r1_ref,
             w0_ref, w1c_ref, wm1_ref, wm2_ref, ys_ref, xg_ref, wrow_ref):
    t = pl.program_id(0)
    f = pl.program_id(1)

    for hb in range(M // HM):
        rs = pl.ds(hb * HM, HM)
        on = nh_ref[t] > hb

        @pl.when(jnp.logical_and(on, f == 0))
        def _(rs=rs, hb=hb):
            rid = (lax.broadcasted_iota(jnp.int32, (HM, T), 0)
                   + t * M + hb * HM)
            oh0 = (rid == r0_ref[0]).astype(jnp.bfloat16)
            oh1 = (rid == r1_ref[0]).astype(jnp.bfloat16)
            xg = lax.dot_general(oh0 + oh1, xnb_ref[...],
                                 (((1,), (0,)), ((), ())),
                                 preferred_element_type=jnp.float32)
            xg_ref[rs, :] = xg
            a = lax.dot_general(oh0, w0_ref[0], (((1,), (0,)), ((), ())),
                                preferred_element_type=jnp.float32)
            b = lax.dot_general(oh1, w1c_ref[0], (((1,), (0,)), ((), ())),
                                preferred_element_type=jnp.float32)
            wrow_ref[rs, :] = jnp.sum(a + b, axis=1, keepdims=True)

        @pl.when(on)
        def _(rs=rs):
            x = xg_ref[rs, :]
            h = lax.dot_general(x, wm1_ref[0], (((1,), (1,)), ((), ())),
                                preferred_element_type=jnp.float32)
            h = h * lax.logistic(h)
            y = lax.dot_general(h, wm2_ref[0], (((1,), (1,)), ((), ())),
                                preferred_element_type=jnp.float32)

            @pl.when(f == 0)
            def _():
                ys_ref[rs, :] = y

            @pl.when(f > 0)
            def _():
                ys_ref[rs, :] += y

        @pl.when(jnp.logical_and(on, f == FC - 1))
        def _(rs=rs):
            ys_ref[rs, :] *= wrow_ref[rs, :]


def _grouped_ffn(xnb, r0, r1, w0cat, w1cat, W1, W2, tile_e, active, blk,
                 n_half):
    grid_spec = pltpu.PrefetchScalarGridSpec(
        num_scalar_prefetch=4,
        grid=(NT, FC),
        in_specs=[
            pl.BlockSpec((T, D), lambda t, f, te, act, blk, nh: (0, 0)),
            pl.BlockSpec((1, 1, T), lambda t, f, te, act, blk, nh: (0, 0, 0)),
            pl.BlockSpec((1, 1, T), lambda t, f, te, act, blk, nh: (0, 0, 0)),
            pl.BlockSpec((1, T, 2), lambda t, f, te, act, blk, nh: (0, 0, 0)),
            pl.BlockSpec((1, T, 2), lambda t, f, te, act, blk, nh: (0, 0, 0)),
            pl.BlockSpec(
                (1, F, D),
                lambda t, f, te, act, blk, nh:
                    (te[t], jnp.where(act[t] == 1, f, FC - 1), 0)),
            pl.BlockSpec(
                (1, D, F),
                lambda t, f, te, act, blk, nh:
                    (te[t], 0, jnp.where(act[t] == 1, f, FC - 1))),
        ],
        out_specs=pl.BlockSpec((M, D), lambda t, f, te, act, blk, nh: (blk[t], 0)),
        scratch_shapes=[pltpu.VMEM((M, D), jnp.float32),
                        pltpu.VMEM((M, 1), jnp.float32)],
    )
    return pl.pallas_call(
        _mm_body,
        grid_spec=grid_spec,
        out_shape=jax.ShapeDtypeStruct((R, D), jnp.float32),
        compiler_params=pltpu.CompilerParams(
            dimension_semantics=("arbitrary", "arbitrary")),
    )(tile_e, active, blk, n_half, xnb, r0.reshape(1, 1, T),
      r1.reshape(1, 1, T), w0cat, w1cat, W1, W2)


CCH = 16


def _combine_body(ys_hbm, r0_hbm, r1_hbm, x_hbm, out_hbm,
                  i0_v, i1_v, a_v, b_v, c_v, sem0, sem1):
    wid = lax.axis_index("s") * NC + lax.axis_index("c")
    per_w = T // NW

    def chunk(i, _):
        base = wid * per_w + i * CCH
        pltpu.sync_copy(r0_hbm.at[pl.ds(base, CCH)], i0_v)
        pltpu.sync_copy(r1_hbm.at[pl.ds(base, CCH)], i1_v)
        cp0 = pltpu.async_copy(ys_hbm.at[i0_v], a_v, sem0)
        cp1 = pltpu.async_copy(ys_hbm.at[i1_v], b_v, sem1)
        pltpu.sync_copy(x_hbm.at[pl.ds(base, CCH)], c_v)
        cp0.wait()
        cp1.wait()

        def row(r, _):
            for j in range(D // LANES):
                s = pl.ds(j * LANES, LANES)
                c_v[r, s] = c_v[r, s] + a_v[r, s] + b_v[r, s]
            return _

        lax.fori_loop(0, CCH, row, None)
        pltpu.sync_copy(c_v, out_hbm.at[pl.ds(base, CCH)])
        return _

    lax.fori_loop(0, per_w // CCH, chunk, None)


def _sc_combine(ys, r0, r1, x2d):
    k = functools.partial(
        pl.kernel,
        out_type=jax.ShapeDtypeStruct((T, D), jnp.float32),
        mesh=_sc_mesh(),
        scratch_types=[
            pltpu.VMEM((CCH,), jnp.int32),
            pltpu.VMEM((CCH,), jnp.int32),
            pltpu.VMEM((CCH, D), jnp.float32),
            pltpu.VMEM((CCH, D), jnp.float32),
            pltpu.VMEM((CCH, D), jnp.float32),
            pltpu.SemaphoreType.DMA,
            pltpu.SemaphoreType.DMA,
        ],
    )(_combine_body)
    return k(ys, r0, r1, x2d)


def kernel(x, ln_g, ln_b, Wr, W1, W2):
    x2d = x.reshape(T, D)
    xnb, probs = _router(x2d, ln_g, ln_b, Wr)
    (w0cat, w1cat, r0, r1, tile_e, active, blk,
     n_half) = _dispatch_metadata(probs)
    ys = _grouped_ffn(xnb, r0, r1, w0cat, w1cat, W1, W2, tile_e, active, blk,
                      n_half)
    out = _sc_combine(ys, r0, r1, x2d)
    return out.reshape(1, T, D)
